# baseline (speedup 1.0000x reference)
import jax
import jax.numpy as jnp
from jax import lax
from jax.experimental import pallas as pl
from jax.experimental.pallas import tpu as pltpu
from jax.experimental.pallas import tpu_sc as plsc


_TOP_K = 2
_CAP_FACTOR = 1.5


def _router_block(x, w1, b1, w2, b2):
    E = w2.shape[1]
    h = jnp.maximum(jnp.dot(x, w1, preferred_element_type=jnp.float32) + b1, 0.0)
    logits = jnp.dot(h, w2, preferred_element_type=jnp.float32) + b2
    m = jnp.max(logits, axis=-1, keepdims=True)
    ex = jnp.exp(logits - m)
    probs = ex / jnp.sum(ex, axis=-1, keepdims=True)

    eidx = jax.lax.broadcasted_iota(jnp.int32, probs.shape, 1)
    m1 = jnp.max(probs, axis=-1, keepdims=True)
    i1 = jnp.min(jnp.where(probs == m1, eidx, E), axis=-1, keepdims=True)
    oh1 = eidx == i1
    pmasked = jnp.where(oh1, -1.0, probs)
    m2 = jnp.max(pmasked, axis=-1, keepdims=True)
    i2 = jnp.min(jnp.where(pmasked == m2, eidx, E), axis=-1, keepdims=True)
    oh2 = eidx == i2
    denom = m1 + m2
    comb_vals = jnp.where(oh1, m1 / denom, jnp.where(oh2, m2 / denom, 0.0))
    disp_vals = jnp.where(oh1 | oh2, 1.0, 0.0)
    return probs, disp_vals, comb_vals


def _tc_body(x_ref, w1_ref, b1_ref, w2_ref, b2_ref,
             disp_ref, cvals_ref, probs_ref, aux_ref, acc_ref):
    i = pl.program_id(0)
    n = pl.num_programs(0)
    T = x_ref.shape[0]
    E = w2_ref.shape[1]
    CAP = disp_ref.shape[2]
    S_total = T * n

    probs, disp_vals, comb_vals = _router_block(
        x_ref[...], w1_ref[...], b1_ref[...], w2_ref[...], b2_ref[...])
    probs_ref[...] = probs
    cvals_ref[...] = comb_vals

    cap0 = jax.lax.broadcasted_iota(jnp.int32, (T, E, 128), 2) == 0
    disp_ref[:, :, 0:128] = jnp.where(cap0, disp_vals[:, :, None], 0.0)
    disp_ref[:, :, 128:CAP] = jnp.zeros((T, E, CAP - 128), dtype=jnp.float32)

    @pl.when(i == 0)
    def _():
        acc_ref[...] = jnp.zeros_like(acc_ref)

    acc_ref[...] += jnp.sum(probs, axis=0, keepdims=True)

    @pl.when(i == n - 1)
    def _():
        rppe = acc_ref[...] / S_total
        aux = jnp.sum(rppe * jnp.log(rppe * E + 1e-9))
        aux_ref[...] = jnp.full((1, 1), aux, dtype=jnp.float32)


_SC_CHUNK = 8


def _make_sc_writer(S, E, CAP):
    info = plsc.get_sparse_core_info()
    NW = info.num_cores * info.num_subcores
    per_w = S // NW
    n_chunks = per_w // _SC_CHUNK
    row_words = E * CAP
    mesh = plsc.VectorSubcoreMesh(core_axis_name="c", subcore_axis_name="s")

    def body(vals_hbm, zeros_hbm, out_hbm, vals_v, buf0, buf1, sem0, sem1):
        cid = lax.axis_index("c")
        sid = lax.axis_index("s")
        wid = sid * info.num_cores + cid
        tok0 = wid * per_w
        pltpu.sync_copy(vals_hbm.at[pl.ds(tok0, per_w)], vals_v)
        pltpu.sync_copy(zeros_hbm, buf0)
        pltpu.sync_copy(zeros_hbm, buf1)
        col_idx = lax.iota(jnp.int32, 16) * CAP
        bufs = (buf0, buf1)
        sems = (sem0, sem1)
        pending = [None, None]
        for c in range(n_chunks):
            b = c % 2
            if pending[b] is not None:
                pending[b].wait()
            for t in range(_SC_CHUNK):
                row = vals_v[c * _SC_CHUNK + t]
                row_idx = jnp.full((16,), t, dtype=jnp.int32)
                plsc.store_scatter(bufs[b], [row_idx, col_idx], row)
            cp = pltpu.make_async_copy(
                bufs[b],
                out_hbm.at[pl.ds(tok0 + c * _SC_CHUNK, _SC_CHUNK)],
                sems[b])
            cp.start()
            pending[b] = cp
        for cp in pending:
            if cp is not None:
                cp.wait()

    return pl.kernel(
        body,
        out_type=jax.ShapeDtypeStruct((S, row_words), jnp.float32),
        mesh=mesh,
        scratch_types=[
            pltpu.VMEM((per_w, E), jnp.float32),
            pltpu.VMEM((_SC_CHUNK, row_words), jnp.float32),
            pltpu.VMEM((_SC_CHUNK, row_words), jnp.float32),
            pltpu.SemaphoreType.DMA,
            pltpu.SemaphoreType.DMA,
        ],
        compiler_params=pltpu.CompilerParams(
            use_tc_tiling_on_sc=False, needs_layout_passes=False),
    )


def kernel(hidden_states, W1, b1, W2, b2):
    B, S, H = hidden_states.shape
    E = W2.shape[1]
    capacity = int(B * S * _CAP_FACTOR * _TOP_K / E)
    T = 256
    grid = (B * S) // T

    x = hidden_states.reshape(B * S, H)
    b1r = b1.reshape(1, H)
    b2r = b2.reshape(1, E)

    disp, cvals, probs, aux = pl.pallas_call(
        _tc_body,
        grid=(grid,),
        in_specs=[
            pl.BlockSpec((T, H), lambda i: (i, 0)),
            pl.BlockSpec((H, H), lambda i: (0, 0)),
            pl.BlockSpec((1, H), lambda i: (0, 0)),
            pl.BlockSpec((H, E), lambda i: (0, 0)),
            pl.BlockSpec((1, E), lambda i: (0, 0)),
        ],
        out_specs=[
            pl.BlockSpec((T, E, capacity), lambda i: (i, 0, 0)),
            pl.BlockSpec((T, E), lambda i: (i, 0)),
            pl.BlockSpec((T, E), lambda i: (i, 0)),
            pl.BlockSpec((1, 1), lambda i: (0, 0)),
        ],
        out_shape=[
            jax.ShapeDtypeStruct((B * S, E, capacity), jnp.float32),
            jax.ShapeDtypeStruct((B * S, E), jnp.float32),
            jax.ShapeDtypeStruct((B * S, E), jnp.float32),
            jax.ShapeDtypeStruct((1, 1), jnp.float32),
        ],
        scratch_shapes=[pltpu.VMEM((1, E), jnp.float32)],
        compiler_params=pltpu.CompilerParams(
            dimension_semantics=("arbitrary",),
        ),
    )(x, W1, b1r, W2, b2r)

    sc_writer = _make_sc_writer(B * S, E, capacity)
    zeros_page = jnp.zeros((_SC_CHUNK, E * capacity), dtype=jnp.float32)
    comb = sc_writer(cvals, zeros_page)

    dispatch = disp.reshape(B, S, E, capacity)
    combine = comb.reshape(B, S, E, capacity)
    router_probs = probs.reshape(B, S, E)
    aux_loss = aux.reshape(())
    return (dispatch, combine, router_probs, aux_loss)


# speedup vs baseline: 1.7850x; 1.7850x over previous
import jax
import jax.numpy as jnp
from jax import lax
from jax.experimental import pallas as pl
from jax.experimental.pallas import tpu as pltpu
from jax.experimental.pallas import tpu_sc as plsc


_TOP_K = 2
_CAP_FACTOR = 1.5


def _router_block(x, w1, b1, w2, b2):
    E = w2.shape[1]
    h = jnp.maximum(jnp.dot(x, w1, preferred_element_type=jnp.float32) + b1, 0.0)
    logits = jnp.dot(h, w2, preferred_element_type=jnp.float32) + b2
    m = jnp.max(logits, axis=-1, keepdims=True)
    ex = jnp.exp(logits - m)
    probs = ex / jnp.sum(ex, axis=-1, keepdims=True)

    eidx = jax.lax.broadcasted_iota(jnp.int32, probs.shape, 1)
    m1 = jnp.max(probs, axis=-1, keepdims=True)
    i1 = jnp.min(jnp.where(probs == m1, eidx, E), axis=-1, keepdims=True)
    oh1 = eidx == i1
    pmasked = jnp.where(oh1, -1.0, probs)
    m2 = jnp.max(pmasked, axis=-1, keepdims=True)
    i2 = jnp.min(jnp.where(pmasked == m2, eidx, E), axis=-1, keepdims=True)
    oh2 = eidx == i2
    denom = m1 + m2
    comb_vals = jnp.where(oh1, m1 / denom, jnp.where(oh2, m2 / denom, 0.0))
    disp_vals = jnp.where(oh1 | oh2, 1.0, 0.0)
    return probs, disp_vals, comb_vals


def _tc_body(x_ref, w1_ref, b1_ref, w2_ref, b2_ref,
             disp_ref, cvals_ref, probs_ref, aux_ref, acc_ref):
    i = pl.program_id(0)
    n = pl.num_programs(0)
    T = x_ref.shape[0]
    E = w2_ref.shape[1]
    CAP = disp_ref.shape[2]
    S_total = T * n

    probs, disp_vals, comb_vals = _router_block(
        x_ref[...], w1_ref[...], b1_ref[...], w2_ref[...], b2_ref[...])
    probs_ref[...] = probs
    cvals_ref[...] = comb_vals

    cap0 = jax.lax.broadcasted_iota(jnp.int32, (T, E, 128), 2) == 0
    disp_ref[:, :, 0:128] = jnp.where(cap0, disp_vals[:, :, None], 0.0)
    disp_ref[:, :, 128:CAP] = jnp.zeros((T, E, CAP - 128), dtype=jnp.float32)

    @pl.when(i == 0)
    def _():
        acc_ref[...] = jnp.zeros_like(acc_ref)

    acc_ref[...] += jnp.sum(probs, axis=0, keepdims=True)

    @pl.when(i == n - 1)
    def _():
        rppe = acc_ref[...] / S_total
        aux = jnp.sum(rppe * jnp.log(rppe * E + 1e-9))
        aux_ref[...] = jnp.full((1, 1), aux, dtype=jnp.float32)


def _panel_body(cvals_ref, _comb_in_ref, comb_ref):
    T, E = cvals_ref.shape
    cap0 = jax.lax.broadcasted_iota(jnp.int32, (T, E, 128), 2) == 0
    comb_ref[:, :, 0:128] = jnp.where(cap0, cvals_ref[...][:, :, None], 0.0)


_SC_CHUNK = 16


def _make_sc_zeros(S, E, CAP):
    info = plsc.get_sparse_core_info()
    NW = info.num_cores * info.num_subcores
    per_w = S // NW
    n_chunks = per_w // _SC_CHUNK
    mesh = plsc.VectorSubcoreMesh(core_axis_name="c", subcore_axis_name="s")

    def body(zeros_hbm, out_hbm, buf, sem):
        cid = lax.axis_index("c")
        sid = lax.axis_index("s")
        wid = sid * info.num_cores + cid
        tok0 = wid * per_w
        pltpu.sync_copy(zeros_hbm, buf)
        pending = []
        for c in range(n_chunks):
            cp = pltpu.make_async_copy(
                buf,
                out_hbm.at[pl.ds(tok0 + c * _SC_CHUNK, _SC_CHUNK)],
                sem)
            cp.start()
            pending.append(cp)
        for cp in pending:
            cp.wait()

    return pl.kernel(
        body,
        out_type=jax.ShapeDtypeStruct((S, E, CAP), jnp.float32),
        mesh=mesh,
        scratch_types=[
            pltpu.VMEM((_SC_CHUNK, E, CAP), jnp.float32),
            pltpu.SemaphoreType.DMA,
        ],
        compiler_params=pltpu.CompilerParams(use_tc_tiling_on_sc=True),
    )


def kernel(hidden_states, W1, b1, W2, b2):
    B, S, H = hidden_states.shape
    E = W2.shape[1]
    capacity = int(B * S * _CAP_FACTOR * _TOP_K / E)
    T = 256
    grid = (B * S) // T

    x = hidden_states.reshape(B * S, H)
    b1r = b1.reshape(1, H)
    b2r = b2.reshape(1, E)

    sc_zeros = _make_sc_zeros(B * S, E, capacity)
    zeros_page = jnp.zeros((_SC_CHUNK, E, capacity), dtype=jnp.float32)
    comb_zeroed = sc_zeros(zeros_page)

    disp, cvals, probs, aux = pl.pallas_call(
        _tc_body,
        grid=(grid,),
        in_specs=[
            pl.BlockSpec((T, H), lambda i: (i, 0)),
            pl.BlockSpec((H, H), lambda i: (0, 0)),
            pl.BlockSpec((1, H), lambda i: (0, 0)),
            pl.BlockSpec((H, E), lambda i: (0, 0)),
            pl.BlockSpec((1, E), lambda i: (0, 0)),
        ],
        out_specs=[
            pl.BlockSpec((T, E, capacity), lambda i: (i, 0, 0)),
            pl.BlockSpec((T, E), lambda i: (i, 0)),
            pl.BlockSpec((T, E), lambda i: (i, 0)),
            pl.BlockSpec((1, 1), lambda i: (0, 0)),
        ],
        out_shape=[
            jax.ShapeDtypeStruct((B * S, E, capacity), jnp.float32),
            jax.ShapeDtypeStruct((B * S, E), jnp.float32),
            jax.ShapeDtypeStruct((B * S, E), jnp.float32),
            jax.ShapeDtypeStruct((1, 1), jnp.float32),
        ],
        scratch_shapes=[pltpu.VMEM((1, E), jnp.float32)],
        compiler_params=pltpu.CompilerParams(
            dimension_semantics=("arbitrary",),
        ),
    )(x, W1, b1r, W2, b2r)

    comb = pl.pallas_call(
        _panel_body,
        grid=(grid,),
        in_specs=[
            pl.BlockSpec((T, E), lambda i: (i, 0)),
            pl.BlockSpec(memory_space=pl.ANY),
        ],
        out_specs=pl.BlockSpec((T, E, 128), lambda i: (i, 0, 0)),
        out_shape=jax.ShapeDtypeStruct((B * S, E, capacity), jnp.float32),
        input_output_aliases={1: 0},
        compiler_params=pltpu.CompilerParams(
            dimension_semantics=("arbitrary",),
        ),
    )(cvals, comb_zeroed)

    dispatch = disp.reshape(B, S, E, capacity)
    combine = comb.reshape(B, S, E, capacity)
    router_probs = probs.reshape(B, S, E)
    aux_loss = aux.reshape(())
    return (dispatch, combine, router_probs, aux_loss)


# speedup vs baseline: 1.9954x; 1.1179x over previous
import jax
import jax.numpy as jnp
from jax import lax
from jax.experimental import pallas as pl
from jax.experimental.pallas import tpu as pltpu
from jax.experimental.pallas import tpu_sc as plsc


_TOP_K = 2
_CAP_FACTOR = 1.5


def _router_block(x, w1, b1, w2, b2):
    E = w2.shape[1]
    h = jnp.maximum(jnp.dot(x, w1, preferred_element_type=jnp.float32) + b1, 0.0)
    logits = jnp.dot(h, w2, preferred_element_type=jnp.float32) + b2
    m = jnp.max(logits, axis=-1, keepdims=True)
    ex = jnp.exp(logits - m)
    probs = ex / jnp.sum(ex, axis=-1, keepdims=True)

    eidx = jax.lax.broadcasted_iota(jnp.int32, probs.shape, 1)
    m1 = jnp.max(probs, axis=-1, keepdims=True)
    i1 = jnp.min(jnp.where(probs == m1, eidx, E), axis=-1, keepdims=True)
    oh1 = eidx == i1
    pmasked = jnp.where(oh1, -1.0, probs)
    m2 = jnp.max(pmasked, axis=-1, keepdims=True)
    i2 = jnp.min(jnp.where(pmasked == m2, eidx, E), axis=-1, keepdims=True)
    oh2 = eidx == i2
    denom = m1 + m2
    comb_vals = jnp.where(oh1, m1 / denom, jnp.where(oh2, m2 / denom, 0.0))
    disp_vals = jnp.where(oh1 | oh2, 1.0, 0.0)
    return probs, disp_vals, comb_vals


def _tc_body(x_ref, w1_ref, b1_ref, w2_ref, b2_ref,
             disp_ref, cvals_ref, probs_ref, aux_ref, acc_ref):
    i = pl.program_id(0)
    n = pl.num_programs(0)
    T = x_ref.shape[0]
    E = w2_ref.shape[1]
    CAP = disp_ref.shape[2]
    S_total = T * n

    probs, disp_vals, comb_vals = _router_block(
        x_ref[...], w1_ref[...], b1_ref[...], w2_ref[...], b2_ref[...])
    probs_ref[...] = probs
    cvals_ref[...] = comb_vals

    cap0 = jax.lax.broadcasted_iota(jnp.int32, (T, E, 128), 2) == 0
    disp_ref[:, :, 0:128] = jnp.where(cap0, disp_vals[:, :, None], 0.0)
    disp_ref[:, :, 128:CAP] = jnp.zeros((T, E, CAP - 128), dtype=jnp.float32)

    @pl.when(i == 0)
    def _():
        acc_ref[...] = jnp.zeros_like(acc_ref)

    acc_ref[...] += jnp.sum(probs, axis=0, keepdims=True)

    @pl.when(i == n - 1)
    def _():
        rppe = acc_ref[...] / S_total
        aux = jnp.sum(rppe * jnp.log(rppe * E + 1e-9))
        aux_ref[...] = jnp.full((1, 1), aux, dtype=jnp.float32)


def _panel_body(cvals_ref, _comb_in_ref, comb_ref):
    T, E = cvals_ref.shape
    cap0 = jax.lax.broadcasted_iota(jnp.int32, (T, E, 128), 2) == 0
    comb_ref[:, :, 0:128] = jnp.where(cap0, cvals_ref[...][:, :, None], 0.0)


_SC_CHUNK = 16


def _make_sc_zeros(S, E, CAP):
    info = plsc.get_sparse_core_info()
    NW = info.num_cores * info.num_subcores
    per_w = S // NW
    n_chunks = per_w // _SC_CHUNK
    mesh = plsc.VectorSubcoreMesh(core_axis_name="c", subcore_axis_name="s")

    def body(zeros_hbm, out_hbm, buf, sem):
        cid = lax.axis_index("c")
        sid = lax.axis_index("s")
        wid = sid * info.num_cores + cid
        tok0 = wid * per_w
        pltpu.sync_copy(zeros_hbm, buf)
        pending = []
        for c in range(n_chunks):
            cp = pltpu.make_async_copy(
                buf,
                out_hbm.at[pl.ds(tok0 + c * _SC_CHUNK, _SC_CHUNK), :,
                           pl.ds(128, CAP - 128)],
                sem)
            cp.start()
            pending.append(cp)
        for cp in pending:
            cp.wait()

    return pl.kernel(
        body,
        out_type=jax.ShapeDtypeStruct((S, E, CAP), jnp.float32),
        mesh=mesh,
        scratch_types=[
            pltpu.VMEM((_SC_CHUNK, E, CAP - 128), jnp.float32),
            pltpu.SemaphoreType.DMA,
        ],
        compiler_params=pltpu.CompilerParams(use_tc_tiling_on_sc=True),
    )


def kernel(hidden_states, W1, b1, W2, b2):
    B, S, H = hidden_states.shape
    E = W2.shape[1]
    capacity = int(B * S * _CAP_FACTOR * _TOP_K / E)
    T = 256
    grid = (B * S) // T

    x = hidden_states.reshape(B * S, H)
    b1r = b1.reshape(1, H)
    b2r = b2.reshape(1, E)

    sc_zeros = _make_sc_zeros(B * S, E, capacity)
    zeros_page = jnp.zeros((_SC_CHUNK, E, capacity - 128), dtype=jnp.float32)
    comb_zeroed = sc_zeros(zeros_page)

    disp, cvals, probs, aux = pl.pallas_call(
        _tc_body,
        grid=(grid,),
        in_specs=[
            pl.BlockSpec((T, H), lambda i: (i, 0)),
            pl.BlockSpec((H, H), lambda i: (0, 0)),
            pl.BlockSpec((1, H), lambda i: (0, 0)),
            pl.BlockSpec((H, E), lambda i: (0, 0)),
            pl.BlockSpec((1, E), lambda i: (0, 0)),
        ],
        out_specs=[
            pl.BlockSpec((T, E, capacity), lambda i: (i, 0, 0)),
            pl.BlockSpec((T, E), lambda i: (i, 0)),
            pl.BlockSpec((T, E), lambda i: (i, 0)),
            pl.BlockSpec((1, 1), lambda i: (0, 0)),
        ],
        out_shape=[
            jax.ShapeDtypeStruct((B * S, E, capacity), jnp.float32),
            jax.ShapeDtypeStruct((B * S, E), jnp.float32),
            jax.ShapeDtypeStruct((B * S, E), jnp.float32),
            jax.ShapeDtypeStruct((1, 1), jnp.float32),
        ],
        scratch_shapes=[pltpu.VMEM((1, E), jnp.float32)],
        compiler_params=pltpu.CompilerParams(
            dimension_semantics=("arbitrary",),
        ),
    )(x, W1, b1r, W2, b2r)

    comb = pl.pallas_call(
        _panel_body,
        grid=(grid,),
        in_specs=[
            pl.BlockSpec((T, E), lambda i: (i, 0)),
            pl.BlockSpec(memory_space=pl.ANY),
        ],
        out_specs=pl.BlockSpec((T, E, 128), lambda i: (i, 0, 0)),
        out_shape=jax.ShapeDtypeStruct((B * S, E, capacity), jnp.float32),
        input_output_aliases={1: 0},
        compiler_params=pltpu.CompilerParams(
            dimension_semantics=("arbitrary",),
        ),
    )(cvals, comb_zeroed)

    dispatch = disp.reshape(B, S, E, capacity)
    combine = comb.reshape(B, S, E, capacity)
    router_probs = probs.reshape(B, S, E)
    aux_loss = aux.reshape(())
    return (dispatch, combine, router_probs, aux_loss)


# speedup vs baseline: 2.0362x; 1.0205x over previous
import jax
import jax.numpy as jnp
from jax import lax
from jax.experimental import pallas as pl
from jax.experimental.pallas import tpu as pltpu
from jax.experimental.pallas import tpu_sc as plsc


_TOP_K = 2
_CAP_FACTOR = 1.5


def _router_block(x, w1, b1, w2, b2):
    E = w2.shape[1]
    h = jnp.maximum(jnp.dot(x, w1, preferred_element_type=jnp.float32) + b1, 0.0)
    logits = jnp.dot(h, w2, preferred_element_type=jnp.float32) + b2
    m = jnp.max(logits, axis=-1, keepdims=True)
    ex = jnp.exp(logits - m)
    probs = ex / jnp.sum(ex, axis=-1, keepdims=True)

    eidx = jax.lax.broadcasted_iota(jnp.int32, probs.shape, 1)
    m1 = jnp.max(probs, axis=-1, keepdims=True)
    i1 = jnp.min(jnp.where(probs == m1, eidx, E), axis=-1, keepdims=True)
    oh1 = eidx == i1
    pmasked = jnp.where(oh1, -1.0, probs)
    m2 = jnp.max(pmasked, axis=-1, keepdims=True)
    i2 = jnp.min(jnp.where(pmasked == m2, eidx, E), axis=-1, keepdims=True)
    oh2 = eidx == i2
    denom = m1 + m2
    comb_vals = jnp.where(oh1, m1 / denom, jnp.where(oh2, m2 / denom, 0.0))
    disp_vals = jnp.where(oh1 | oh2, 1.0, 0.0)
    return probs, disp_vals, comb_vals


def _tc_body(x_ref, w1_ref, b1_ref, w2_ref, b2_ref,
             disp_ref, cvals_ref, probs_ref, aux_ref, acc_ref):
    i = pl.program_id(0)
    n = pl.num_programs(0)
    T = x_ref.shape[0]
    E = w2_ref.shape[1]
    CAP = disp_ref.shape[2]
    S_total = T * n

    probs, disp_vals, comb_vals = _router_block(
        x_ref[...], w1_ref[...], b1_ref[...], w2_ref[...], b2_ref[...])
    probs_ref[...] = probs
    cvals_ref[...] = comb_vals

    cap0 = jax.lax.broadcasted_iota(jnp.int32, (T, E, 128), 2) == 0
    disp_ref[:, :, 0:128] = jnp.where(cap0, disp_vals[:, :, None], 0.0)
    disp_ref[:, :, 128:CAP] = jnp.zeros((T, E, CAP - 128), dtype=jnp.float32)

    @pl.when(i == 0)
    def _():
        acc_ref[...] = jnp.zeros_like(acc_ref)

    acc_ref[...] += jnp.sum(probs, axis=0, keepdims=True)

    @pl.when(i == n - 1)
    def _():
        rppe = acc_ref[...] / S_total
        aux = jnp.sum(rppe * jnp.log(rppe * E + 1e-9))
        aux_ref[...] = jnp.full((1, 1), aux, dtype=jnp.float32)


_CSPLIT = 256


def _panel_body(cvals_ref, _comb_in_ref, comb_ref):
    T, E = cvals_ref.shape
    cap0 = jax.lax.broadcasted_iota(jnp.int32, (T, E, _CSPLIT), 2) == 0
    comb_ref[:, :, 0:_CSPLIT] = jnp.where(cap0, cvals_ref[...][:, :, None], 0.0)


_SC_CHUNK = 16


def _make_sc_zeros(S, E, CAP):
    info = plsc.get_sparse_core_info()
    NW = info.num_cores * info.num_subcores
    per_w = S // NW
    n_chunks = per_w // _SC_CHUNK
    mesh = plsc.VectorSubcoreMesh(core_axis_name="c", subcore_axis_name="s")

    def body(zeros_hbm, out_hbm, buf, sem):
        cid = lax.axis_index("c")
        sid = lax.axis_index("s")
        wid = sid * info.num_cores + cid
        tok0 = wid * per_w
        pltpu.sync_copy(zeros_hbm, buf)
        pending = []
        for c in range(n_chunks):
            cp = pltpu.make_async_copy(
                buf,
                out_hbm.at[pl.ds(tok0 + c * _SC_CHUNK, _SC_CHUNK), :,
                           pl.ds(_CSPLIT, CAP - _CSPLIT)],
                sem)
            cp.start()
            pending.append(cp)
        for cp in pending:
            cp.wait()

    return pl.kernel(
        body,
        out_type=jax.ShapeDtypeStruct((S, E, CAP), jnp.float32),
        mesh=mesh,
        scratch_types=[
            pltpu.VMEM((_SC_CHUNK, E, CAP - _CSPLIT), jnp.float32),
            pltpu.SemaphoreType.DMA,
        ],
        compiler_params=pltpu.CompilerParams(use_tc_tiling_on_sc=True),
    )


def kernel(hidden_states, W1, b1, W2, b2):
    B, S, H = hidden_states.shape
    E = W2.shape[1]
    capacity = int(B * S * _CAP_FACTOR * _TOP_K / E)
    T = 256
    grid = (B * S) // T

    x = hidden_states.reshape(B * S, H)
    b1r = b1.reshape(1, H)
    b2r = b2.reshape(1, E)

    sc_zeros = _make_sc_zeros(B * S, E, capacity)
    zeros_page = jnp.zeros((_SC_CHUNK, E, capacity - _CSPLIT), dtype=jnp.float32)
    comb_zeroed = sc_zeros(zeros_page)

    disp, cvals, probs, aux = pl.pallas_call(
        _tc_body,
        grid=(grid,),
        in_specs=[
            pl.BlockSpec((T, H), lambda i: (i, 0)),
            pl.BlockSpec((H, H), lambda i: (0, 0)),
            pl.BlockSpec((1, H), lambda i: (0, 0)),
            pl.BlockSpec((H, E), lambda i: (0, 0)),
            pl.BlockSpec((1, E), lambda i: (0, 0)),
        ],
        out_specs=[
            pl.BlockSpec((T, E, capacity), lambda i: (i, 0, 0)),
            pl.BlockSpec((T, E), lambda i: (i, 0)),
            pl.BlockSpec((T, E), lambda i: (i, 0)),
            pl.BlockSpec((1, 1), lambda i: (0, 0)),
        ],
        out_shape=[
            jax.ShapeDtypeStruct((B * S, E, capacity), jnp.float32),
            jax.ShapeDtypeStruct((B * S, E), jnp.float32),
            jax.ShapeDtypeStruct((B * S, E), jnp.float32),
            jax.ShapeDtypeStruct((1, 1), jnp.float32),
        ],
        scratch_shapes=[pltpu.VMEM((1, E), jnp.float32)],
        compiler_params=pltpu.CompilerParams(
            dimension_semantics=("arbitrary",),
        ),
    )(x, W1, b1r, W2, b2r)

    comb = pl.pallas_call(
        _panel_body,
        grid=(grid,),
        in_specs=[
            pl.BlockSpec((T, E), lambda i: (i, 0)),
            pl.BlockSpec(memory_space=pl.ANY),
        ],
        out_specs=pl.BlockSpec((T, E, _CSPLIT), lambda i: (i, 0, 0)),
        out_shape=jax.ShapeDtypeStruct((B * S, E, capacity), jnp.float32),
        input_output_aliases={1: 0},
        compiler_params=pltpu.CompilerParams(
            dimension_semantics=("arbitrary",),
        ),
    )(cvals, comb_zeroed)

    dispatch = disp.reshape(B, S, E, capacity)
    combine = comb.reshape(B, S, E, capacity)
    router_probs = probs.reshape(B, S, E)
    aux_loss = aux.reshape(())
    return (dispatch, combine, router_probs, aux_loss)
